# baseline (speedup 1.0000x reference)
import functools

import jax
import jax.numpy as jnp
from jax import lax
from jax.experimental import pallas as pl
from jax.experimental.pallas import tpu as pltpu
from jax.experimental.pallas import tpu_sc as plsc

EMBED_DIM = 64
BATCH = 4096
LANES = 16


@functools.cache
def _make_gather(B, D, V):
    info = plsc.get_sparse_core_info()
    NW = info.num_cores * info.num_subcores
    n_blocks = (V + 127) // 128
    last_full = (V // 128)
    last_w = V - last_full * 128
    bpw = (n_blocks + NW - 1) // NW
    n_vecs = B // LANES
    mesh = plsc.VectorSubcoreMesh(core_axis_name="c", subcore_axis_name="s")

    @functools.partial(
        pl.kernel,
        mesh=mesh,
        out_type=jax.ShapeDtypeStruct((B * D,), jnp.float32),
        compiler_params=pltpu.CompilerParams(
            skip_device_barrier=True,
            needs_layout_passes=False,
        ),
        scratch_types=[
            pltpu.VMEM((B,), jnp.int32),
            pltpu.VMEM((B,), jnp.int32),
            pltpu.VMEM((B,), jnp.int32),
            pltpu.VMEM((2, D, 128), jnp.float32),
            pltpu.VMEM((16, D), jnp.float32),
            pltpu.VMEM((LANES,), jnp.int32),
            pltpu.VMEM((LANES,), jnp.int32),
            pltpu.SemaphoreType.DMA,
            pltpu.SemaphoreType.DMA,
        ],
    )
    def gather_kernel(tab_t, tail_t, idx_hbm, out_hbm, idx_v, list_i, list_b,
                      blk, colring, st_i, st_b, sem_blk, sem_col):
        wid = lax.axis_index("s") * info.num_cores + lax.axis_index("c")
        lo = wid * bpw
        hi = jnp.minimum(lo + bpw, n_blocks)
        iota = lax.iota(jnp.int32, LANES)

        pltpu.sync_copy(idx_hbm, idx_v)

        @pl.loop(0, n_vecs, init_carry=jnp.int32(0))
        def _scan(g, count):
            vec = idx_v[pl.ds(g * LANES, LANES)]
            bv = g * LANES + iota
            blks = lax.shift_right_logical(vec, 7)
            inr = (blks >= lo) & (blks < hi)
            cnt = plsc.all_reduce_population_count(inr)[0]
            plsc.store_compressed(st_i.at[:], vec, mask=inr)
            plsc.store_compressed(st_b.at[:], bv, mask=inr)
            posv = count + iota
            m2 = iota < cnt
            plsc.store_scatter(list_i.at[:], [posv], st_i[...], mask=m2)
            plsc.store_scatter(list_b.at[:], [posv], st_b[...], mask=m2)
            return count + cnt

        count = _scan
        ngv = lax.div(count + (LANES - 1), LANES)

        def process_block(blk2d, bid, w, fired0):
            @pl.loop(0, ngv, init_carry=fired0)
            def _pb(g2, fired):
                lv = list_i[pl.ds(g2 * LANES, LANES)]
                bv = list_b[pl.ds(g2 * LANES, LANES)]
                valid = (g2 * LANES + iota) < count
                m = valid & (lax.shift_right_logical(lv, 7) == bid)
                c2 = plsc.all_reduce_population_count(m)[0]
                plsc.store_compressed(st_i.at[:], lv, mask=m)
                plsc.store_compressed(st_b.at[:], bv, mask=m)
                si = st_i[...]
                sb = st_b[...]
                for j in range(LANES):
                    fired_j = fired + j

                    @pl.when(j < c2)
                    def _hit():
                        lane = si[j] & 127
                        b = sb[j]
                        ring = lax.rem(fired_j, 16)

                        @pl.when(fired_j >= 16)
                        def _ring_wait():
                            pltpu.make_async_copy(
                                colring.at[0], out_hbm.at[pl.ds(0, D)], sem_col
                            ).wait()

                        lvec = jnp.full((LANES,), lane, jnp.int32)
                        for g3 in range(D // LANES):
                            cv = g3 * LANES + iota
                            vals = plsc.load_gather(blk2d, [cv, lvec])
                            colring[ring, pl.ds(g3 * LANES, LANES)] = vals
                        pltpu.make_async_copy(
                            colring.at[ring],
                            out_hbm.at[pl.ds(pl.multiple_of(b * D, D), D)],
                            sem_col,
                        ).start()

                return fired + c2

            return _pb

        fired = jnp.int32(0)

        @pl.when(hi > last_full)
        def _tail_load():
            pltpu.sync_copy(tail_t, blk.at[0])

        fired = lax.cond(
            hi > last_full,
            lambda: process_block(blk.at[0], jnp.int32(last_full), wid, fired),
            lambda: fired,
        )

        nb = jnp.minimum(hi, last_full) - lo

        @pl.when(nb > 0)
        def _prime():
            pltpu.make_async_copy(
                tab_t.at[:, pl.ds(pl.multiple_of(lo * 128, 128), 128)],
                blk.at[0],
                sem_blk,
            ).start()

        @pl.loop(0, nb, init_carry=fired)
        def _blocks(k, fired_c):
            slot = lax.rem(k, 2)

            @pl.when(k + 1 < nb)
            def _prefetch():
                pltpu.make_async_copy(
                    tab_t.at[:, pl.ds(pl.multiple_of((lo + k + 1) * 128, 128), 128)],
                    blk.at[lax.rem(k + 1, 2)],
                    sem_blk,
                ).start()

            pltpu.make_async_copy(
                tab_t.at[:, pl.ds(0, 128)], blk.at[0], sem_blk
            ).wait()
            return process_block(blk.at[slot], lo + k, wid, fired_c)

        total_fired = _blocks if _blocks is not None else fired

        @pl.loop(0, jnp.minimum(total_fired, 16))
        def _drain(_):
            pltpu.make_async_copy(
                colring.at[0], out_hbm.at[pl.ds(0, D)], sem_col
            ).wait()

    return gather_kernel


def kernel(user_id, table):
    idx = user_id.astype(jnp.int32)
    V, D = table.shape
    B = user_id.shape[0]
    last_full = V // 128
    tail = table[last_full * 128:]
    tail_t = jnp.pad(tail, ((0, 128 - tail.shape[0]), (0, 0))).T
    out_flat = _make_gather(B, D, V)(table.T, tail_t, idx)
    return out_flat.reshape(B, D)


# speedup vs baseline: 2.0643x; 2.0643x over previous
import functools

import jax
import jax.numpy as jnp
from jax import lax
from jax.experimental import pallas as pl
from jax.experimental.pallas import tpu as pltpu
from jax.experimental.pallas import tpu_sc as plsc

EMBED_DIM = 64
BATCH = 4096
LANES = 16
RB = 13


@functools.cache
def _make_gather(B, D, V):
    info = plsc.get_sparse_core_info()
    NW = info.num_cores * info.num_subcores
    n_blocks = (V + 127) // 128
    last_full = V // 128
    bpw = (n_blocks + NW - 1) // NW
    n_rounds = (bpw + RB - 1) // RB
    n_vecs = B // LANES
    mesh = plsc.VectorSubcoreMesh(core_axis_name="c", subcore_axis_name="s")

    @functools.partial(
        pl.kernel,
        mesh=mesh,
        out_type=jax.ShapeDtypeStruct(((B + 1) * D,), jnp.float32),
        compiler_params=pltpu.CompilerParams(
            skip_device_barrier=True,
            needs_layout_passes=False,
        ),
        scratch_types=[
            pltpu.VMEM((B,), jnp.int32),
            pltpu.VMEM((B,), jnp.int32),
            pltpu.VMEM((RB, D, 128), jnp.float32),
            pltpu.VMEM((LANES, D), jnp.float32),
            pltpu.VMEM((LANES,), jnp.int32),
            pltpu.SemaphoreType.DMA,
            pltpu.SemaphoreType.DMA,
        ],
    )
    def gather_kernel(tab_t, tail_t, idx_hbm, out_hbm, idx_v, listp, bigblk,
                      stage, st, sem_blk, sem_col):
        wid = lax.axis_index("s") * info.num_cores + lax.axis_index("c")
        lo = wid * bpw
        hi = jnp.minimum(lo + bpw, n_blocks)
        iota = lax.iota(jnp.int32, LANES)

        pltpu.sync_copy(idx_hbm, idx_v)

        @pl.loop(0, n_vecs, init_carry=jnp.int32(0))
        def _scan(g, count):
            vec = idx_v[pl.ds(g * LANES, LANES)]
            blks = lax.shift_right_logical(vec, 7)
            inr = (blks >= lo) & (blks < hi)
            cnt = plsc.all_reduce_population_count(inr)[0]
            packed = vec | ((g * LANES + iota) << 17)
            plsc.store_compressed(st.at[:], packed, mask=inr)
            sv = st[...]
            plsc.store_scatter(listp.at[:], [count + iota], sv, mask=iota < cnt)
            return count + cnt

        count = _scan
        ngv = lax.div(count + (LANES - 1), LANES)

        @pl.loop(0, n_rounds, init_carry=jnp.int32(0))
        def _rounds(r, pend_in):
            base_blk = lo + r * RB
            nfr = jnp.clip(hi - base_blk, 0, RB)

            for kk in range(RB):
                bid = base_blk + kk

                @pl.when(bid < jnp.minimum(hi, last_full))
                def _load():
                    pltpu.make_async_copy(
                        tab_t.at[:, pl.ds(pl.multiple_of(bid * 128, 128), 128)],
                        bigblk.at[kk],
                        sem_blk,
                    ).start()

                @pl.when((bid == last_full) & (hi > last_full))
                def _load_tail():
                    pltpu.make_async_copy(tail_t, bigblk.at[kk], sem_blk).start()

            @pl.loop(0, nfr)
            def _drain_blk(_):
                pltpu.make_async_copy(
                    tab_t.at[:, pl.ds(0, 128)], bigblk.at[0], sem_blk
                ).wait()

            @pl.loop(0, ngv, init_carry=pend_in)
            def _groups(g2, pend):
                pv0 = listp[pl.ds(g2 * LANES, LANES)]
                valid = (g2 * LANES + iota) < count
                idxv = pv0 & 0x1FFFF
                bb = lax.shift_right_logical(pv0, 17)
                bs = lax.shift_right_logical(idxv, 7) - base_blk
                m = valid & (bs >= 0) & (bs < nfr)
                c2 = plsc.all_reduce_population_count(m)[0]

                @pl.loop(0, pend)
                def _drain_col(_):
                    pltpu.make_async_copy(
                        stage.at[0], out_hbm.at[pl.ds(0, D)], sem_col
                    ).wait()

                @pl.when(c2 > 0)
                def _proc():
                    bsc = jnp.clip(bs, 0, RB - 1)
                    lane = idxv & 127
                    rows = jnp.where(m, bb, jnp.int32(B))
                    for c in range(D):
                        cvec = jnp.full((LANES,), c, jnp.int32)
                        vals = plsc.load_gather(bigblk.at[:], [bsc, cvec, lane])
                        plsc.store_scatter(stage.at[:], [iota, cvec], vals)
                    for j in range(LANES):
                        rj = rows[j]

                        @pl.when(rj < B)
                        def _fire():
                            pltpu.make_async_copy(
                                stage.at[j],
                                out_hbm.at[pl.ds(pl.multiple_of(rj * D, D), D)],
                                sem_col,
                            ).start()

                return c2

            return _groups

        pend_final = _rounds

        @pl.loop(0, pend_final)
        def _drain_last(_):
            pltpu.make_async_copy(
                stage.at[0], out_hbm.at[pl.ds(0, D)], sem_col
            ).wait()

    return gather_kernel


def kernel(user_id, table):
    idx = user_id.astype(jnp.int32)
    V, D = table.shape
    B = user_id.shape[0]
    last_full = V // 128
    tail = table[last_full * 128:]
    tail_t = jnp.pad(tail, ((0, 128 - tail.shape[0]), (0, 0))).T
    out_flat = _make_gather(B, D, V)(table.T, tail_t, idx)
    return out_flat[: B * D].reshape(B, D)


# speedup vs baseline: 2.1508x; 1.0419x over previous
import functools

import jax
import jax.numpy as jnp
from jax import lax
from jax.experimental import pallas as pl
from jax.experimental.pallas import tpu as pltpu
from jax.experimental.pallas import tpu_sc as plsc

EMBED_DIM = 64
BATCH = 4096
LANES = 16
RB = 13


@functools.cache
def _make_gather(B, D, V):
    info = plsc.get_sparse_core_info()
    NW = info.num_cores * info.num_subcores
    n_blocks = (V + 127) // 128
    last_full = V // 128
    bpw = (n_blocks + NW - 1) // NW
    n_rounds = (bpw + RB - 1) // RB
    n_vecs = B // LANES
    mesh = plsc.VectorSubcoreMesh(core_axis_name="c", subcore_axis_name="s")

    @functools.partial(
        pl.kernel,
        mesh=mesh,
        out_type=jax.ShapeDtypeStruct(((B + 1) * D,), jnp.float32),
        compiler_params=pltpu.CompilerParams(
            skip_device_barrier=True,
            needs_layout_passes=False,
        ),
        scratch_types=[
            pltpu.VMEM((B,), jnp.int32),
            pltpu.VMEM((B,), jnp.int32),
            pltpu.VMEM((RB, D, 128), jnp.float32),
            pltpu.VMEM((LANES, D), jnp.float32),
            pltpu.VMEM((LANES,), jnp.int32),
            pltpu.SemaphoreType.DMA,
            pltpu.SemaphoreType.DMA,
        ],
    )
    def gather_kernel(tab_t, tail_t, idx_hbm, out_hbm, idx_v, listp, bigblk,
                      stage, st, sem_blk, sem_col):
        wid = lax.axis_index("s") * info.num_cores + lax.axis_index("c")
        lo = wid * bpw
        hi = jnp.minimum(lo + bpw, n_blocks)
        iota = lax.iota(jnp.int32, LANES)

        for kk0 in range(RB):
            bid0 = lo + kk0

            @pl.when(bid0 < jnp.minimum(hi, last_full))
            def _load0():
                pltpu.make_async_copy(
                    tab_t.at[:, pl.ds(pl.multiple_of(bid0 * 128, 128), 128)],
                    bigblk.at[kk0],
                    sem_blk,
                ).start()

            @pl.when((bid0 == last_full) & (hi > last_full))
            def _load0_tail():
                pltpu.make_async_copy(tail_t, bigblk.at[kk0], sem_blk).start()

        pltpu.sync_copy(idx_hbm, idx_v)

        @pl.loop(0, n_vecs, init_carry=jnp.int32(0), unroll=4)
        def _scan(g, count):
            vec = idx_v[pl.ds(g * LANES, LANES)]
            blks = lax.shift_right_logical(vec, 7)
            inr = (blks >= lo) & (blks < hi)
            cnt = plsc.all_reduce_population_count(inr)[0]
            packed = vec | ((g * LANES + iota) << 17)
            plsc.store_compressed(st.at[:], packed, mask=inr)
            sv = st[...]
            plsc.store_scatter(listp.at[:], [count + iota], sv, mask=iota < cnt)
            return count + cnt

        count = _scan
        ngv = lax.div(count + (LANES - 1), LANES)

        @pl.loop(0, n_rounds, init_carry=jnp.int32(0))
        def _rounds(r, pend_in):
            base_blk = lo + r * RB
            nfr = jnp.clip(hi - base_blk, 0, RB)

            @pl.when(r > 0)
            def _loads():
                for kk in range(RB):
                    bid = base_blk + kk

                    @pl.when(bid < jnp.minimum(hi, last_full))
                    def _load():
                        pltpu.make_async_copy(
                            tab_t.at[
                                :, pl.ds(pl.multiple_of(bid * 128, 128), 128)
                            ],
                            bigblk.at[kk],
                            sem_blk,
                        ).start()

                    @pl.when((bid == last_full) & (hi > last_full))
                    def _load_tail():
                        pltpu.make_async_copy(
                            tail_t, bigblk.at[kk], sem_blk
                        ).start()

            @pl.loop(0, nfr)
            def _drain_blk(_):
                pltpu.make_async_copy(
                    tab_t.at[:, pl.ds(0, 128)], bigblk.at[0], sem_blk
                ).wait()

            @pl.loop(0, ngv, init_carry=pend_in)
            def _groups(g2, pend):
                pv0 = listp[pl.ds(g2 * LANES, LANES)]
                valid = (g2 * LANES + iota) < count
                idxv = pv0 & 0x1FFFF
                bb = lax.shift_right_logical(pv0, 17)
                bs = lax.shift_right_logical(idxv, 7) - base_blk
                m = valid & (bs >= 0) & (bs < nfr)
                c2 = plsc.all_reduce_population_count(m)[0]

                @pl.loop(0, pend)
                def _drain_col(_):
                    pltpu.make_async_copy(
                        stage.at[0], out_hbm.at[pl.ds(0, D)], sem_col
                    ).wait()

                @pl.when(c2 > 0)
                def _proc():
                    bsc = jnp.clip(bs, 0, RB - 1)
                    lane = idxv & 127
                    rows = jnp.where(m, bb, jnp.int32(B))
                    for c in range(D):
                        cvec = jnp.full((LANES,), c, jnp.int32)
                        vals = plsc.load_gather(bigblk.at[:], [bsc, cvec, lane])
                        plsc.store_scatter(stage.at[:], [iota, cvec], vals)
                    for j in range(LANES):
                        rj = rows[j]

                        @pl.when(rj < B)
                        def _fire():
                            pltpu.make_async_copy(
                                stage.at[j],
                                out_hbm.at[pl.ds(pl.multiple_of(rj * D, D), D)],
                                sem_col,
                            ).start()

                return c2

            return _groups

        pend_final = _rounds

        @pl.loop(0, pend_final)
        def _drain_last(_):
            pltpu.make_async_copy(
                stage.at[0], out_hbm.at[pl.ds(0, D)], sem_col
            ).wait()

    return gather_kernel


def kernel(user_id, table):
    idx = user_id.astype(jnp.int32)
    V, D = table.shape
    B = user_id.shape[0]
    last_full = V // 128
    tail = table[last_full * 128:]
    tail_t = jnp.pad(tail, ((0, 128 - tail.shape[0]), (0, 0))).T
    out_flat = _make_gather(B, D, V)(table.T, tail_t, idx)
    return out_flat[: B * D].reshape(B, D)


# speedup vs baseline: 2.4218x; 1.1260x over previous
import functools

import jax
import jax.numpy as jnp
from jax import lax
from jax.experimental import pallas as pl
from jax.experimental.pallas import tpu as pltpu
from jax.experimental.pallas import tpu_sc as plsc

EMBED_DIM = 64
BATCH = 4096
LANES = 16
RB = 13


@functools.cache
def _make_gather(B, D, V):
    info = plsc.get_sparse_core_info()
    NW = info.num_cores * info.num_subcores
    n_blocks = (V + 127) // 128
    last_full = V // 128
    bpw = (n_blocks + NW - 1) // NW
    assert bpw <= 2 * RB, "two-round structure assumes bpw <= 2*RB"
    n_vecs = B // LANES
    mesh = plsc.VectorSubcoreMesh(core_axis_name="c", subcore_axis_name="s")

    @functools.partial(
        pl.kernel,
        mesh=mesh,
        out_type=jax.ShapeDtypeStruct(((B + 1) * D,), jnp.float32),
        compiler_params=pltpu.CompilerParams(
            skip_device_barrier=True,
            needs_layout_passes=False,
        ),
        scratch_types=[
            pltpu.VMEM((B,), jnp.int32),
            pltpu.VMEM((B,), jnp.int32),
            pltpu.VMEM((B,), jnp.int32),
            pltpu.VMEM((RB, D, 128), jnp.float32),
            pltpu.VMEM((LANES, D), jnp.float32),
            pltpu.VMEM((LANES,), jnp.int32),
            pltpu.SemaphoreType.DMA,
            pltpu.SemaphoreType.DMA,
        ],
    )
    def gather_kernel(tab_t, tail_t, idx_hbm, out_hbm, idx_v, list0, list1,
                      bigblk, stage, st, sem_blk, sem_col):
        wid = lax.axis_index("s") * info.num_cores + lax.axis_index("c")
        lo = wid * bpw
        hi = jnp.minimum(lo + bpw, n_blocks)
        iota = lax.iota(jnp.int32, LANES)
        mid = jnp.minimum(lo + RB, hi)

        def fire_loads(base_blk):
            for kk in range(RB):
                bid = base_blk + kk

                @pl.when(bid < jnp.minimum(hi, last_full))
                def _load():
                    pltpu.make_async_copy(
                        tab_t.at[:, pl.ds(pl.multiple_of(bid * 128, 128), 128)],
                        bigblk.at[kk],
                        sem_blk,
                    ).start()

                @pl.when((bid == last_full) & (hi > last_full))
                def _load_tail():
                    pltpu.make_async_copy(tail_t, bigblk.at[kk], sem_blk).start()

        fire_loads(lo)
        pltpu.sync_copy(idx_hbm, idx_v)

        @pl.loop(0, n_vecs, init_carry=(jnp.int32(0), jnp.int32(0)), unroll=4)
        def _scan(g, counts):
            c0, c1 = counts
            vec = idx_v[pl.ds(g * LANES, LANES)]
            blks = lax.shift_right_logical(vec, 7)
            packed = vec | ((g * LANES + iota) << 17)
            in0 = (blks >= lo) & (blks < mid)
            in1 = (blks >= mid) & (blks < hi)
            n0 = plsc.all_reduce_population_count(in0)[0]
            n1 = plsc.all_reduce_population_count(in1)[0]
            plsc.store_compressed(st.at[:], packed, mask=in0)
            sv0 = st[...]
            plsc.store_scatter(list0.at[:], [c0 + iota], sv0, mask=iota < n0)
            plsc.store_compressed(st.at[:], packed, mask=in1)
            sv1 = st[...]
            plsc.store_scatter(list1.at[:], [c1 + iota], sv1, mask=iota < n1)
            return (c0 + n0, c1 + n1)

        count0, count1 = _scan

        def run_round(lst, count, base_blk, nfr, pend_in):
            ngv = lax.div(count + (LANES - 1), LANES)

            @pl.loop(0, nfr)
            def _drain_blk(_):
                pltpu.make_async_copy(
                    tab_t.at[:, pl.ds(0, 128)], bigblk.at[0], sem_blk
                ).wait()

            @pl.loop(0, ngv, init_carry=pend_in)
            def _groups(g2, pend):
                pv0 = lst[pl.ds(g2 * LANES, LANES)]
                m = (g2 * LANES + iota) < count
                idxv = pv0 & 0x1FFFF
                bb = lax.shift_right_logical(pv0, 17)
                bs = lax.shift_right_logical(idxv, 7) - base_blk
                c2 = plsc.all_reduce_population_count(m)[0]

                @pl.loop(0, pend)
                def _drain_col(_):
                    pltpu.make_async_copy(
                        stage.at[0], out_hbm.at[pl.ds(0, D)], sem_col
                    ).wait()

                bsc = jnp.clip(bs, 0, RB - 1)
                lane = idxv & 127
                rows = jnp.where(m, bb, jnp.int32(B))
                for c in range(D):
                    cvec = jnp.full((LANES,), c, jnp.int32)
                    vals = plsc.load_gather(bigblk.at[:], [bsc, cvec, lane])
                    plsc.store_scatter(stage.at[:], [iota, cvec], vals)
                for j in range(LANES):
                    rj = rows[j]

                    @pl.when(rj < B)
                    def _fire():
                        pltpu.make_async_copy(
                            stage.at[j],
                            out_hbm.at[pl.ds(pl.multiple_of(rj * D, D), D)],
                            sem_col,
                        ).start()

                return c2

            return _groups

        nfr0 = jnp.clip(mid - lo, 0, RB)
        pend = run_round(list0, count0, lo, nfr0, jnp.int32(0))

        fire_loads(lo + RB)
        nfr1 = jnp.clip(hi - (lo + RB), 0, RB)
        pend = run_round(list1, count1, lo + RB, nfr1, pend)

        @pl.loop(0, pend)
        def _drain_last(_):
            pltpu.make_async_copy(
                stage.at[0], out_hbm.at[pl.ds(0, D)], sem_col
            ).wait()

    return gather_kernel


def kernel(user_id, table):
    idx = user_id.astype(jnp.int32)
    V, D = table.shape
    B = user_id.shape[0]
    last_full = V // 128
    tail = table[last_full * 128:]
    tail_t = jnp.pad(tail, ((0, 128 - tail.shape[0]), (0, 0))).T
    out_flat = _make_gather(B, D, V)(table.T, tail_t, idx)
    return out_flat[: B * D].reshape(B, D)
